# baseline (speedup 1.0000x reference)
import functools

import jax
import jax.numpy as jnp
from jax import lax
from jax.experimental import pallas as pl
from jax.experimental.pallas import tpu as pltpu
from jax.experimental.pallas import tpu_sc as plsc

N = 10000
E = 320000
D = 128

NC = 2
NS = 16
NW = NC * NS
EPW = E // NW
CHUNK = 80
NCHUNK = EPW // CHUNK
DTILES = 10
DR = N // DTILES
ZR = 40


def _sc_segment_sum(x, edge_flat):
    mesh = plsc.VectorSubcoreMesh(core_axis_name="c", subcore_axis_name="s")

    @functools.partial(
        pl.kernel,
        mesh=mesh,
        out_type=jax.ShapeDtypeStruct((NC, N, D), jnp.float32),
        scratch_types=[
            pltpu.VMEM((EPW,), jnp.int32),
            pltpu.VMEM((EPW,), jnp.int32),
            pltpu.VMEM((2, CHUNK), jnp.int32),
            pltpu.VMEM((2, CHUNK, D), jnp.float32),
            pltpu.VMEM((ZR, D), jnp.float32),
            pltpu.VMEM_SHARED((N, D), jnp.float32),
            pltpu.SemaphoreType.DMA,
            [pltpu.SemaphoreType.DMA] * 2,
            pltpu.SemaphoreType.DMA,
        ],
    )
    def body(x_hbm, ei_hbm, out_hbm, src_v, dst_v, sidx_v, rows_v, zero_v,
             acc_sh, psem, gsems, zsem):
        c = lax.axis_index("c")
        s = lax.axis_index("s")
        wid = s * NC + c
        base_w = wid * EPW

        cp_s = pltpu.make_async_copy(
            ei_hbm.at[pl.ds(base_w, EPW)], src_v, psem)
        cp_s.start()
        cp_d = pltpu.make_async_copy(
            ei_hbm.at[pl.ds(E + base_w, EPW)], dst_v, psem)
        cp_d.start()

        def gather_start(ci, b):
            pltpu.make_async_copy(
                x_hbm.at[src_v.at[pl.ds(ci * CHUNK, CHUNK)]],
                rows_v.at[b], gsems[b]).start()

        def gather_wait(b):
            pltpu.make_async_copy(
                x_hbm.at[src_v.at[pl.ds(0, CHUNK)]],
                rows_v.at[b], gsems[b]).wait()

        @pl.when(s < DTILES)
        def _():
            def zrow(i, carry):
                def zcol(jj, carry2):
                    zero_v[i, pl.ds(jj * 16, 16)] = jnp.zeros((16,),
                                                              jnp.float32)
                    return carry2
                return lax.fori_loop(0, D // 16, zcol, carry)
            lax.fori_loop(0, ZR, zrow, 0)
            for z in range(DR // ZR):
                pltpu.make_async_copy(
                    zero_v, acc_sh.at[pl.ds(s * DR + z * ZR, ZR)],
                    zsem).start()
            for z in range(DR // ZR):
                pltpu.make_async_copy(
                    zero_v, acc_sh.at[pl.ds(s * DR + z * ZR, ZR)],
                    zsem).wait()

        cp_s.wait()
        cp_d.wait()
        gather_start(0, 0)
        gather_start(1, 1)

        plsc.subcore_barrier()

        def step(ci, b, guard):
            gather_wait(b)
            def cp(i, carry):
                sidx_v[b, pl.ds(i * 16, 16)] = dst_v[pl.ds(
                    ci * CHUNK + i * 16, 16)]
                return carry
            lax.fori_loop(0, CHUNK // 16, cp, 0)
            pltpu.sync_copy(rows_v.at[b], acc_sh.at[sidx_v.at[b]], add=True)
            if guard:
                @pl.when(ci + 2 < NCHUNK)
                def _():
                    gather_start(ci + 2, b)
            else:
                gather_start(ci + 2, b)

        def pair(k, carry):
            step(2 * k, 0, False)
            step(2 * k + 1, 1, False)
            return carry
        lax.fori_loop(0, (NCHUNK - 3) // 2, pair, 0)
        step(NCHUNK - 3, 0, True)
        step(NCHUNK - 2, 1, True)
        step(NCHUNK - 1, 0, True)

        plsc.subcore_barrier()

        @pl.when(s < DTILES)
        def _():
            pltpu.sync_copy(acc_sh.at[pl.ds(s * DR, DR)],
                            out_hbm.at[c, pl.ds(s * DR, DR)])

    return body(x, edge_flat)


BLK = 1000


def _tc_self(x, weight_self):
    def body(x_ref, ws_ref, o_ref):
        o_ref[...] = jnp.dot(x_ref[...], ws_ref[...],
                             preferred_element_type=jnp.float32)

    return pl.pallas_call(
        body,
        grid=(N // BLK,),
        in_specs=[
            pl.BlockSpec((BLK, D), lambda i: (i, 0)),
            pl.BlockSpec((D, D), lambda i: (0, 0)),
        ],
        out_specs=pl.BlockSpec((BLK, D), lambda i: (i, 0)),
        out_shape=jax.ShapeDtypeStruct((N, D), jnp.float32),
    )(x, weight_self)


def _tc_combine(part, xs, weight):
    def body(p_ref, xs_ref, w_ref, o_ref):
        agg = p_ref[0] + p_ref[1]
        o_ref[...] = (
            jnp.dot(agg, w_ref[...], preferred_element_type=jnp.float32)
            + xs_ref[...]
        )

    return pl.pallas_call(
        body,
        grid=(N // BLK,),
        in_specs=[
            pl.BlockSpec((NC, BLK, D), lambda i: (0, i, 0)),
            pl.BlockSpec((BLK, D), lambda i: (i, 0)),
            pl.BlockSpec((D, D), lambda i: (0, 0)),
        ],
        out_specs=pl.BlockSpec((BLK, D), lambda i: (i, 0)),
        out_shape=jax.ShapeDtypeStruct((N, D), jnp.float32),
    )(part, xs, weight)


def kernel(x, edge_index, weight, weight_self):
    xs = _tc_self(x, weight_self)
    part = _sc_segment_sum(x, edge_index.reshape(-1))
    return _tc_combine(part, xs, weight)


# speedup vs baseline: 1.0171x; 1.0171x over previous
import functools

import jax
import jax.numpy as jnp
from jax import lax
from jax.experimental import pallas as pl
from jax.experimental.pallas import tpu as pltpu
from jax.experimental.pallas import tpu_sc as plsc

N = 10000
E = 320000
D = 128

NC = 2
NS = 16
NW = NC * NS
EPW = E // NW
CHUNK = 80
NCHUNK = EPW // CHUNK
DTILES = 10
DR = N // DTILES
ZR = 40


def _sc_segment_sum(x, edge_flat):
    mesh = plsc.VectorSubcoreMesh(core_axis_name="c", subcore_axis_name="s")

    @functools.partial(
        pl.kernel,
        mesh=mesh,
        out_type=jax.ShapeDtypeStruct((NC, N, D), jnp.float32),
        scratch_types=[
            pltpu.VMEM((4, CHUNK), jnp.int32),
            pltpu.VMEM((4, CHUNK), jnp.int32),
            pltpu.VMEM((4, CHUNK), jnp.int32),
            pltpu.VMEM((4, CHUNK, D), jnp.float32),
            pltpu.VMEM((ZR, D), jnp.float32),
            pltpu.VMEM_SHARED((N, D), jnp.float32),
            [pltpu.SemaphoreType.DMA] * 4,
            [pltpu.SemaphoreType.DMA] * 4,
            [pltpu.SemaphoreType.DMA] * 4,
            pltpu.SemaphoreType.DMA,
        ],
    )
    def body(x_hbm, ei_hbm, out_hbm, src_v, dst_v, sidx_v, rows_v, zero_v,
             acc_sh, isems, gsems, ssems, zsem):
        c = lax.axis_index("c")
        s = lax.axis_index("s")
        wid = s * NC + c
        base_w = wid * EPW

        def idx_load(ci, j):
            q = j % 4
            off = base_w + ci * CHUNK
            pltpu.make_async_copy(
                ei_hbm.at[pl.ds(off, CHUNK)], src_v.at[q],
                isems[q]).start()
            pltpu.make_async_copy(
                ei_hbm.at[pl.ds(E + off, CHUNK)], dst_v.at[q],
                isems[q]).start()

        def idx_wait(j):
            q = j % 4
            pltpu.make_async_copy(
                ei_hbm.at[pl.ds(0, CHUNK)], src_v.at[q], isems[q]).wait()
            pltpu.make_async_copy(
                ei_hbm.at[pl.ds(0, CHUNK)], dst_v.at[q], isems[q]).wait()

        def gather_start(j):
            q = j % 4
            pltpu.make_async_copy(
                x_hbm.at[src_v.at[q]], rows_v.at[q], gsems[q]).start()

        def gather_wait(j):
            q = j % 4
            pltpu.make_async_copy(
                x_hbm.at[src_v.at[q]], rows_v.at[q], gsems[q]).wait()

        def scat_start(j):
            q = j % 4
            def cp(i, carry):
                sidx_v[q, pl.ds(i * 16, 16)] = dst_v[q, pl.ds(i * 16, 16)]
                return carry
            lax.fori_loop(0, CHUNK // 16, cp, 0)
            pltpu.make_async_copy(
                rows_v.at[q], acc_sh.at[sidx_v.at[q]],
                ssems[q]).start(add=True)

        def scat_wait(j):
            q = j % 4
            pltpu.make_async_copy(
                rows_v.at[q], acc_sh.at[sidx_v.at[q]], ssems[q]).wait()

        for cj in range(4):
            idx_load(cj, cj)

        @pl.when(s < DTILES)
        def _():
            def zrow(i, carry):
                def zcol(jj, carry2):
                    zero_v[i, pl.ds(jj * 16, 16)] = jnp.zeros((16,),
                                                              jnp.float32)
                    return carry2
                return lax.fori_loop(0, D // 16, zcol, carry)
            lax.fori_loop(0, ZR, zrow, 0)
            for z in range(DR // ZR):
                pltpu.make_async_copy(
                    zero_v, acc_sh.at[pl.ds(s * DR + z * ZR, ZR)],
                    zsem).start()
            for z in range(DR // ZR):
                pltpu.make_async_copy(
                    zero_v, acc_sh.at[pl.ds(s * DR + z * ZR, ZR)],
                    zsem).wait()

        idx_wait(0)
        gather_start(0)
        idx_wait(1)
        gather_start(1)

        plsc.subcore_barrier()

        def step(ci, j, lo_ok):
            gather_wait(j)
            scat_start(j)
            if lo_ok:
                scat_wait(j + 2)

            @pl.when(ci + 4 < NCHUNK)
            def _():
                idx_load(ci + 4, j)

            @pl.when(ci + 2 < NCHUNK)
            def _():
                idx_wait(j + 2)
                gather_start(j + 2)

        for j in range(4):
            step(j, j, j >= 2)

        def quad(k, carry):
            for j in range(4):
                step(4 * k + j, j, True)
            return carry
        lax.fori_loop(1, NCHUNK // 4, quad, 0)

        step(NCHUNK - 1, 0, True)

        scat_wait(NCHUNK - 2)
        scat_wait(NCHUNK - 1)

        plsc.subcore_barrier()

        @pl.when(s < DTILES)
        def _():
            pltpu.sync_copy(acc_sh.at[pl.ds(s * DR, DR)],
                            out_hbm.at[c, pl.ds(s * DR, DR)])

    return body(x, edge_flat)


BLK = 1000


def _tc_self(x, weight_self):
    def body(x_ref, ws_ref, o_ref):
        o_ref[...] = jnp.dot(x_ref[...], ws_ref[...],
                             preferred_element_type=jnp.float32)

    return pl.pallas_call(
        body,
        grid=(N // BLK,),
        in_specs=[
            pl.BlockSpec((BLK, D), lambda i: (i, 0)),
            pl.BlockSpec((D, D), lambda i: (0, 0)),
        ],
        out_specs=pl.BlockSpec((BLK, D), lambda i: (i, 0)),
        out_shape=jax.ShapeDtypeStruct((N, D), jnp.float32),
    )(x, weight_self)


def _tc_combine(part, xs, weight):
    def body(p_ref, xs_ref, w_ref, o_ref):
        agg = p_ref[0] + p_ref[1]
        o_ref[...] = (
            jnp.dot(agg, w_ref[...], preferred_element_type=jnp.float32)
            + xs_ref[...]
        )

    return pl.pallas_call(
        body,
        grid=(N // BLK,),
        in_specs=[
            pl.BlockSpec((NC, BLK, D), lambda i: (0, i, 0)),
            pl.BlockSpec((BLK, D), lambda i: (i, 0)),
            pl.BlockSpec((D, D), lambda i: (0, 0)),
        ],
        out_specs=pl.BlockSpec((BLK, D), lambda i: (i, 0)),
        out_shape=jax.ShapeDtypeStruct((N, D), jnp.float32),
    )(part, xs, weight)


def kernel(x, edge_index, weight, weight_self):
    xs = _tc_self(x, weight_self)
    part = _sc_segment_sum(x, edge_index.reshape(-1))
    return _tc_combine(part, xs, weight)


# speedup vs baseline: 1.0266x; 1.0092x over previous
import functools

import jax
import jax.numpy as jnp
from jax import lax
from jax.experimental import pallas as pl
from jax.experimental.pallas import tpu as pltpu
from jax.experimental.pallas import tpu_sc as plsc

N = 10000
E = 320000
D = 128

NC = 2
NS = 16
NW = NC * NS
EPW = E // NW
CHUNK = 80
NCHUNK = EPW // CHUNK
DR = 624
DRL = N - 15 * DR
ZR = 16


def _sc_segment_sum(x, edge_flat):
    mesh = plsc.VectorSubcoreMesh(core_axis_name="c", subcore_axis_name="s")

    @functools.partial(
        pl.kernel,
        mesh=mesh,
        out_type=jax.ShapeDtypeStruct((NC, N, D), jnp.float32),
        scratch_types=[
            pltpu.VMEM((4, CHUNK), jnp.int32),
            pltpu.VMEM((4, CHUNK), jnp.int32),
            pltpu.VMEM((4, CHUNK), jnp.int32),
            pltpu.VMEM((4, CHUNK, D), jnp.float32),
            pltpu.VMEM((ZR, D), jnp.float32),
            pltpu.VMEM_SHARED((N, D), jnp.float32),
            [pltpu.SemaphoreType.DMA] * 4,
            [pltpu.SemaphoreType.DMA] * 4,
            [pltpu.SemaphoreType.DMA] * 4,
            pltpu.SemaphoreType.DMA,
        ],
    )
    def body(x_hbm, ei_hbm, out_hbm, src_v, dst_v, sidx_v, rows_v, zero_v,
             acc_sh, isems, gsems, ssems, zsem):
        c = lax.axis_index("c")
        s = lax.axis_index("s")
        wid = s * NC + c
        base_w = wid * EPW

        def idx_load(ci, j):
            q = j % 4
            off = base_w + ci * CHUNK
            pltpu.make_async_copy(
                ei_hbm.at[pl.ds(off, CHUNK)], src_v.at[q],
                isems[q]).start()
            pltpu.make_async_copy(
                ei_hbm.at[pl.ds(E + off, CHUNK)], dst_v.at[q],
                isems[q]).start()

        def idx_wait(j):
            q = j % 4
            pltpu.make_async_copy(
                ei_hbm.at[pl.ds(0, CHUNK)], src_v.at[q], isems[q]).wait()
            pltpu.make_async_copy(
                ei_hbm.at[pl.ds(0, CHUNK)], dst_v.at[q], isems[q]).wait()

        def gather_start(j):
            q = j % 4
            pltpu.make_async_copy(
                x_hbm.at[src_v.at[q]], rows_v.at[q], gsems[q]).start()

        def gather_wait(j):
            q = j % 4
            pltpu.make_async_copy(
                x_hbm.at[src_v.at[q]], rows_v.at[q], gsems[q]).wait()

        def scat_start(j):
            q = j % 4
            def cp(i, carry):
                sidx_v[q, pl.ds(i * 16, 16)] = dst_v[q, pl.ds(i * 16, 16)]
                return carry
            lax.fori_loop(0, CHUNK // 16, cp, 0)
            pltpu.make_async_copy(
                rows_v.at[q], acc_sh.at[sidx_v.at[q]],
                ssems[q]).start(add=True)

        def scat_wait(j):
            q = j % 4
            pltpu.make_async_copy(
                rows_v.at[q], acc_sh.at[sidx_v.at[q]], ssems[q]).wait()

        for cj in range(4):
            idx_load(cj, cj)

        def zrow(i, carry):
            def zcol(jj, carry2):
                zero_v[i, pl.ds(jj * 16, 16)] = jnp.zeros((16,),
                                                          jnp.float32)
                return carry2
            return lax.fori_loop(0, D // 16, zcol, carry)
        lax.fori_loop(0, ZR, zrow, 0)

        @pl.when(s < 15)
        def _():
            for z in range(DR // ZR):
                pltpu.make_async_copy(
                    zero_v, acc_sh.at[pl.ds(s * DR + z * ZR, ZR)],
                    zsem).start()
            for z in range(DR // ZR):
                pltpu.make_async_copy(
                    zero_v, acc_sh.at[pl.ds(s * DR + z * ZR, ZR)],
                    zsem).wait()

        @pl.when(s == 15)
        def _():
            for z in range(DRL // ZR):
                pltpu.make_async_copy(
                    zero_v, acc_sh.at[pl.ds(15 * DR + z * ZR, ZR)],
                    zsem).start()
            for z in range(DRL // ZR):
                pltpu.make_async_copy(
                    zero_v, acc_sh.at[pl.ds(15 * DR + z * ZR, ZR)],
                    zsem).wait()

        idx_wait(0)
        gather_start(0)
        idx_wait(1)
        gather_start(1)

        plsc.subcore_barrier()

        def step(ci, j, lo_ok):
            gather_wait(j)
            scat_start(j)
            if lo_ok:
                scat_wait(j + 2)

            @pl.when(ci + 4 < NCHUNK)
            def _():
                idx_load(ci + 4, j)

            @pl.when(ci + 2 < NCHUNK)
            def _():
                idx_wait(j + 2)
                gather_start(j + 2)

        for j in range(4):
            step(j, j, j >= 2)

        def quad(k, carry):
            for j in range(4):
                step(4 * k + j, j, True)
            return carry
        lax.fori_loop(1, NCHUNK // 4, quad, 0)

        step(NCHUNK - 1, 0, True)

        scat_wait(NCHUNK - 2)
        scat_wait(NCHUNK - 1)

        plsc.subcore_barrier()

        @pl.when(s < 15)
        def _():
            pltpu.sync_copy(acc_sh.at[pl.ds(s * DR, DR)],
                            out_hbm.at[c, pl.ds(s * DR, DR)])

        @pl.when(s == 15)
        def _():
            pltpu.sync_copy(acc_sh.at[pl.ds(15 * DR, DRL)],
                            out_hbm.at[c, pl.ds(15 * DR, DRL)])

    return body(x, edge_flat)


BLK = 1000


def _tc_self(x, weight_self):
    def body(x_ref, ws_ref, o_ref):
        o_ref[...] = jnp.dot(x_ref[...], ws_ref[...],
                             preferred_element_type=jnp.float32)

    return pl.pallas_call(
        body,
        grid=(N // BLK,),
        in_specs=[
            pl.BlockSpec((BLK, D), lambda i: (i, 0)),
            pl.BlockSpec((D, D), lambda i: (0, 0)),
        ],
        out_specs=pl.BlockSpec((BLK, D), lambda i: (i, 0)),
        out_shape=jax.ShapeDtypeStruct((N, D), jnp.float32),
    )(x, weight_self)


def _tc_combine(part, xs, weight):
    def body(p_ref, xs_ref, w_ref, o_ref):
        agg = p_ref[0] + p_ref[1]
        o_ref[...] = (
            jnp.dot(agg, w_ref[...], preferred_element_type=jnp.float32)
            + xs_ref[...]
        )

    return pl.pallas_call(
        body,
        grid=(N // BLK,),
        in_specs=[
            pl.BlockSpec((NC, BLK, D), lambda i: (0, i, 0)),
            pl.BlockSpec((BLK, D), lambda i: (i, 0)),
            pl.BlockSpec((D, D), lambda i: (0, 0)),
        ],
        out_specs=pl.BlockSpec((BLK, D), lambda i: (i, 0)),
        out_shape=jax.ShapeDtypeStruct((N, D), jnp.float32),
    )(part, xs, weight)


def kernel(x, edge_index, weight, weight_self):
    xs = _tc_self(x, weight_self)
    part = _sc_segment_sum(x, edge_index.reshape(-1))
    return _tc_combine(part, xs, weight)
